# baseline (speedup 1.0000x reference)
import functools

import jax
import jax.numpy as jnp
from jax import lax
from jax.experimental import pallas as pl
from jax.experimental.pallas import tpu as pltpu
from jax.experimental.pallas import tpu_sc as plsc

B = 4096
L = 200
VOCAB = 100000
EMB_DIM = 64
HIDDEN = 256
OUT = 5

NC, NS = 2, 16
NW = NC * NS
ROWS_PER_W = B // NW
CHUNK = 104
LP = 2 * CHUNK
CHUNKS_PER_ROW = 2
NCHUNKS = B * CHUNKS_PER_ROW
CHUNKS_PER_W = ROWS_PER_W * CHUNKS_PER_ROW
NBUF = 4
NGRP = CHUNKS_PER_W // NBUF
HALF = CHUNK // 2
NH = EMB_DIM // 32

OUT_PAD = 128


def _pool_body(idx_hbm, emb_hbm, out_hbm, idx_v, b0, b1, b2, b3,
               acc_v, s0, s1, s2, s3):
    bufs = (b0, b1, b2, b3)
    sems = (s0, s1, s2, s3)
    wid = lax.axis_index("s") * NC + lax.axis_index("c")
    pltpu.sync_copy(
        idx_hbm.at[pl.ds(wid * CHUNKS_PER_W * CHUNK, CHUNKS_PER_W * CHUNK)],
        idx_v)

    def idxs(c):
        return idx_v.at[pl.ds(pl.multiple_of(c * CHUNK, 8), CHUNK)]

    lanes = lax.iota(jnp.int32, 16)
    cols = [(32 * h + 2 * lanes, 32 * h + 2 * lanes + 1) for h in range(NH)]

    for b in range(NBUF):
        pltpu.async_copy(emb_hbm.at[idxs(b)], bufs[b], sems[b])

    def acc_chunk(buf, chains):
        for t in range(CHUNK):
            for h in range(NH):
                x = buf[t, pl.ds(32 * h, 32)]
                e, o = plsc.unpack(x, format=plsc.PackFormat.INTERLEAVED)
                hv = 0 if t < HALF else 1
                for eo, v in ((0, e), (1, o)):
                    cur = chains[h][eo][hv]
                    chains[h][eo][hv] = v if cur is None else cur + v
        return chains

    def grp_body(g, carry):
        c0 = NBUF * g
        for p in range(NBUF // 2):
            chains = [[[None, None], [None, None]] for _ in range(NH)]
            for q in range(2):
                b = 2 * p + q
                pltpu.make_async_copy(
                    emb_hbm.at[idxs(0)], bufs[b], sems[b]).wait()
                chains = acc_chunk(bufs[b], chains)

                @pl.when(g < NGRP - 1)
                def _():
                    pltpu.async_copy(
                        emb_hbm.at[idxs(c0 + b + NBUF)], bufs[b], sems[b])
            r = (NBUF // 2) * g + p
            rvec = jnp.full((16,), 0, jnp.int32) + r
            for h in range(NH):
                for eo in range(2):
                    tot = chains[h][eo][0] + chains[h][eo][1]
                    plsc.store_scatter(acc_v, [rvec, cols[h][eo]], tot)
        return carry

    lax.fori_loop(0, NGRP, grp_body, 0)
    pltpu.sync_copy(acc_v, out_hbm.at[pl.ds(wid * ROWS_PER_W, ROWS_PER_W)])


@functools.lru_cache(maxsize=1)
def _make_pool():
    return pl.kernel(
        _pool_body,
        out_type=jax.ShapeDtypeStruct((B, EMB_DIM), jnp.float32),
        mesh=plsc.VectorSubcoreMesh(core_axis_name="c", subcore_axis_name="s"),
        compiler_params=pltpu.CompilerParams(
            use_tc_tiling_on_sc=False, needs_layout_passes=False),
        scratch_types=(
            [pltpu.VMEM((CHUNKS_PER_W * CHUNK,), jnp.int32)]
            + [pltpu.VMEM((CHUNK, EMB_DIM), jnp.bfloat16)
               for _ in range(NBUF)]
            + [pltpu.VMEM((ROWS_PER_W, EMB_DIM), jnp.float32)]
            + [pltpu.SemaphoreType.DMA for _ in range(NBUF)]
        ),
    )


def _mlp_body(x_ref, w1t_ref, b1_ref, w2t_ref, b2_ref, o_ref):
    x = x_ref[:] * (1.0 / L)
    h = jnp.dot(x, w1t_ref[:], preferred_element_type=jnp.float32) + b1_ref[:]
    h = jnp.maximum(h, 0.0)
    o = jnp.dot(h, w2t_ref[:], preferred_element_type=jnp.float32) + b2_ref[:]
    m = jnp.max(o, axis=1, keepdims=True)
    lse = jnp.log(jnp.sum(jnp.exp(o - m), axis=1, keepdims=True)) + m
    o_ref[:] = o - lse


def _mlp(sums, w1t, b1_2d, w2tp, b2p):
    blk = B // 4
    return pl.pallas_call(
        _mlp_body,
        grid=(4,),
        in_specs=[
            pl.BlockSpec((blk, EMB_DIM), lambda i: (i, 0)),
            pl.BlockSpec((EMB_DIM, HIDDEN), lambda i: (0, 0)),
            pl.BlockSpec((1, HIDDEN), lambda i: (0, 0)),
            pl.BlockSpec((HIDDEN, OUT_PAD), lambda i: (0, 0)),
            pl.BlockSpec((1, OUT_PAD), lambda i: (0, 0)),
        ],
        out_specs=pl.BlockSpec((blk, OUT_PAD), lambda i: (i, 0)),
        out_shape=jax.ShapeDtypeStruct((B, OUT_PAD), jnp.float32),
    )(sums, w1t, b1_2d, w2tp, b2p)


def kernel(subword_indices, emb, W1, b1, W2, b2):
    idx = subword_indices.astype(jnp.int32)
    idx = jnp.pad(idx, ((0, 0), (0, LP - L))).reshape(-1)
    sums = _make_pool()(idx, emb.astype(jnp.bfloat16))
    w1t = W1.T
    b1_2d = b1.reshape(1, HIDDEN)
    w2tp = jnp.zeros((HIDDEN, OUT_PAD), jnp.float32).at[:, :OUT].set(W2.T)
    b2p = jnp.full((1, OUT_PAD), -1e30, jnp.float32).at[0, :OUT].set(b2)
    out = _mlp(sums, w1t, b1_2d, w2tp, b2p)
    return out[:, :OUT]


# speedup vs baseline: 2.5189x; 2.5189x over previous
import functools

import jax
import jax.numpy as jnp
from jax import lax
from jax.experimental import pallas as pl
from jax.experimental.pallas import tpu as pltpu
from jax.experimental.pallas import tpu_sc as plsc

B = 4096
L = 200
VOCAB = 100000
EMB_DIM = 64
HIDDEN = 256
OUT = 5

NC, NS = 2, 16
NW = NC * NS
ROWS_PER_W = B // NW
CHUNK = 100
CHUNKS_PER_ROW = L // CHUNK
NCHUNKS = B * CHUNKS_PER_ROW
CHUNKS_PER_W = ROWS_PER_W * CHUNKS_PER_ROW
NBUF = 4
NGRP = CHUNKS_PER_W // NBUF
HALF = CHUNK // 2
NH = EMB_DIM // 32

VROWS_PER_W = VOCAB // NW
CROWS = 125
NCGRP = VROWS_PER_W // CROWS

OUT_PAD = 128

_SC_PARAMS = dict(
    mesh=plsc.VectorSubcoreMesh(core_axis_name="c", subcore_axis_name="s"),
    compiler_params=pltpu.CompilerParams(
        use_tc_tiling_on_sc=False, needs_layout_passes=False),
)


def _conv_body(emb_hbm, out_hbm, f0, f1, o0, o1, s0, s1, t0, t1):
    fbufs = (f0, f1)
    obufs = (o0, o1)
    isems = (s0, s1)
    osems = (t0, t1)
    wid = lax.axis_index("s") * NC + lax.axis_index("c")
    rbase = wid * VROWS_PER_W
    for b in range(2):
        pltpu.async_copy(
            emb_hbm.at[pl.ds(rbase + b * CROWS, CROWS)], fbufs[b], isems[b])

    def pack_chunk(b):
        for r in range(CROWS):
            for h in range(NH):
                a = fbufs[b][r, pl.ds(32 * h, 16)]
                c = fbufs[b][r, pl.ds(32 * h + 16, 16)]
                obufs[b][r, pl.ds(32 * h, 32)] = plsc.pack(
                    a, c, format=plsc.PackFormat.INTERLEAVED)

    def grp_body(k, carry):
        for b in range(2):
            kk = 2 * k + b
            pltpu.make_async_copy(
                emb_hbm.at[pl.ds(0, CROWS)], fbufs[b], isems[b]).wait()

            @pl.when(k > 0)
            def _():
                pltpu.make_async_copy(
                    obufs[b], out_hbm.at[pl.ds(0, CROWS)], osems[b]).wait()
            pack_chunk(b)
            pltpu.async_copy(
                obufs[b], out_hbm.at[pl.ds(rbase + kk * CROWS, CROWS)],
                osems[b])

            @pl.when(kk + 2 < NCGRP)
            def _():
                pltpu.async_copy(
                    emb_hbm.at[pl.ds(rbase + (kk + 2) * CROWS, CROWS)],
                    fbufs[b], isems[b])
        return carry

    lax.fori_loop(0, NCGRP // 2, grp_body, 0)
    pltpu.make_async_copy(
        emb_hbm.at[pl.ds(0, CROWS)], fbufs[0], isems[0]).wait()
    pltpu.make_async_copy(
        obufs[0], out_hbm.at[pl.ds(0, CROWS)], osems[0]).wait()
    pack_chunk(0)
    pltpu.async_copy(
        obufs[0], out_hbm.at[pl.ds(rbase + (NCGRP - 1) * CROWS, CROWS)],
        osems[0])
    pltpu.make_async_copy(
        obufs[0], out_hbm.at[pl.ds(0, CROWS)], osems[0]).wait()
    pltpu.make_async_copy(
        obufs[1], out_hbm.at[pl.ds(0, CROWS)], osems[1]).wait()


@functools.lru_cache(maxsize=1)
def _make_conv():
    return pl.kernel(
        _conv_body,
        out_type=jax.ShapeDtypeStruct((VOCAB, EMB_DIM), jnp.bfloat16),
        scratch_types=(
            [pltpu.VMEM((CROWS, EMB_DIM), jnp.float32) for _ in range(2)]
            + [pltpu.VMEM((CROWS, EMB_DIM), jnp.bfloat16) for _ in range(2)]
            + [pltpu.SemaphoreType.DMA for _ in range(4)]
        ),
        **_SC_PARAMS,
    )


def _pool_body(idx_hbm, emb_hbm, out_hbm, idx_v, b0, b1, b2, b3,
               acc_v, s0, s1, s2, s3):
    bufs = (b0, b1, b2, b3)
    sems = (s0, s1, s2, s3)
    wid = lax.axis_index("s") * NC + lax.axis_index("c")
    cbase = wid * CHUNKS_PER_W
    pltpu.sync_copy(idx_hbm.at[pl.ds(cbase, CHUNKS_PER_W)], idx_v)

    for b in range(NBUF):
        pltpu.async_copy(emb_hbm.at[idx_v.at[b]], bufs[b], sems[b])

    def acc_chunk(buf, chains):
        for t in range(CHUNK):
            for h in range(NH):
                x = buf[t, pl.ds(32 * h, 32)]
                a, c = plsc.unpack(x, format=plsc.PackFormat.INTERLEAVED)
                hv = 0 if t < HALF else 1
                for ab, v in ((0, a), (1, c)):
                    cur = chains[h][ab][hv]
                    chains[h][ab][hv] = v if cur is None else cur + v
        return chains

    def grp_body(g, carry):
        c0 = NBUF * g
        for p in range(NBUF // 2):
            chains = [[[None, None], [None, None]] for _ in range(NH)]
            for q in range(2):
                b = 2 * p + q
                pltpu.make_async_copy(
                    emb_hbm.at[idx_v.at[0]], bufs[b], sems[b]).wait()
                chains = acc_chunk(bufs[b], chains)

                @pl.when(g < NGRP - 1)
                def _():
                    pltpu.async_copy(
                        emb_hbm.at[idx_v.at[c0 + b + NBUF]], bufs[b], sems[b])
            r = (NBUF // 2) * g + p
            for h in range(NH):
                for ab in range(2):
                    tot = chains[h][ab][0] + chains[h][ab][1]
                    acc_v[r, pl.ds(32 * h + 16 * ab, 16)] = tot
        return carry

    lax.fori_loop(0, NGRP, grp_body, 0)
    pltpu.sync_copy(acc_v, out_hbm.at[pl.ds(wid * ROWS_PER_W, ROWS_PER_W)])


@functools.lru_cache(maxsize=1)
def _make_pool():
    return pl.kernel(
        _pool_body,
        out_type=jax.ShapeDtypeStruct((B, EMB_DIM), jnp.float32),
        scratch_types=(
            [pltpu.VMEM((CHUNKS_PER_W, CHUNK), jnp.int32)]
            + [pltpu.VMEM((CHUNK, EMB_DIM), jnp.bfloat16)
               for _ in range(NBUF)]
            + [pltpu.VMEM((ROWS_PER_W, EMB_DIM), jnp.float32)]
            + [pltpu.SemaphoreType.DMA for _ in range(NBUF)]
        ),
        **_SC_PARAMS,
    )


def _mlp_body(x_ref, w1t_ref, b1_ref, w2t_ref, b2_ref, o_ref):
    x = x_ref[:] * (1.0 / L)
    h = jnp.dot(x, w1t_ref[:], preferred_element_type=jnp.float32) + b1_ref[:]
    h = jnp.maximum(h, 0.0)
    o = jnp.dot(h, w2t_ref[:], preferred_element_type=jnp.float32) + b2_ref[:]
    m = jnp.max(o, axis=1, keepdims=True)
    lse = jnp.log(jnp.sum(jnp.exp(o - m), axis=1, keepdims=True)) + m
    o_ref[:] = o - lse


def _mlp(sums, w1t, b1_2d, w2tp, b2p):
    blk = B // 4
    return pl.pallas_call(
        _mlp_body,
        grid=(4,),
        in_specs=[
            pl.BlockSpec((blk, EMB_DIM), lambda i: (i, 0)),
            pl.BlockSpec((EMB_DIM, HIDDEN), lambda i: (0, 0)),
            pl.BlockSpec((1, HIDDEN), lambda i: (0, 0)),
            pl.BlockSpec((HIDDEN, OUT_PAD), lambda i: (0, 0)),
            pl.BlockSpec((1, OUT_PAD), lambda i: (0, 0)),
        ],
        out_specs=pl.BlockSpec((blk, OUT_PAD), lambda i: (i, 0)),
        out_shape=jax.ShapeDtypeStruct((B, OUT_PAD), jnp.float32),
    )(sums, w1t, b1_2d, w2tp, b2p)


def kernel(subword_indices, emb, W1, b1, W2, b2):
    idx = subword_indices.astype(jnp.int32).reshape(NCHUNKS, CHUNK)
    emb_packed = _make_conv()(emb)
    sums = _make_pool()(idx, emb_packed)
    w1t = W1.T
    b1_2d = b1.reshape(1, HIDDEN)
    w2tp = jnp.zeros((HIDDEN, OUT_PAD), jnp.float32).at[:, :OUT].set(W2.T)
    b2p = jnp.full((1, OUT_PAD), -1e30, jnp.float32).at[0, :OUT].set(b2)
    out = _mlp(sums, w1t, b1_2d, w2tp, b2p)
    return out[:, :OUT]


# speedup vs baseline: 2.5241x; 1.0021x over previous
import functools

import jax
import jax.numpy as jnp
from jax import lax
from jax.experimental import pallas as pl
from jax.experimental.pallas import tpu as pltpu
from jax.experimental.pallas import tpu_sc as plsc

B = 4096
L = 200
VOCAB = 100000
EMB_DIM = 64
HIDDEN = 256
OUT = 5

NC, NS = 2, 16
NW = NC * NS
ROWS_PER_W = B // NW
CHUNK = 100
CHUNKS_PER_ROW = L // CHUNK
NCHUNKS = B * CHUNKS_PER_ROW
CHUNKS_PER_W = ROWS_PER_W * CHUNKS_PER_ROW
NBUF = 4
NGRP = CHUNKS_PER_W // NBUF
HALF = CHUNK // 2
NH = EMB_DIM // 32

VROWS_PER_W = VOCAB // NW
CROWS = 125
NCGRP = VROWS_PER_W // CROWS

OUT_PAD = 128

_SC_PARAMS = dict(
    mesh=plsc.VectorSubcoreMesh(core_axis_name="c", subcore_axis_name="s"),
    compiler_params=pltpu.CompilerParams(
        use_tc_tiling_on_sc=False, needs_layout_passes=False),
)


def _conv_body(emb_hbm, out_hbm, f0, f1, o0, o1, s0, s1, t0, t1):
    fbufs = (f0, f1)
    obufs = (o0, o1)
    isems = (s0, s1)
    osems = (t0, t1)
    wid = lax.axis_index("s") * NC + lax.axis_index("c")
    rbase = wid * VROWS_PER_W
    for b in range(2):
        pltpu.async_copy(
            emb_hbm.at[pl.ds(rbase + b * CROWS, CROWS)], fbufs[b], isems[b])

    def pack_chunk(b):
        for r in range(CROWS):
            for h in range(NH):
                a = fbufs[b][r, pl.ds(32 * h, 16)]
                c = fbufs[b][r, pl.ds(32 * h + 16, 16)]
                obufs[b][r, pl.ds(32 * h, 32)] = plsc.pack(
                    a, c, format=plsc.PackFormat.INTERLEAVED)

    def grp_body(k, carry):
        for b in range(2):
            kk = 2 * k + b
            pltpu.make_async_copy(
                emb_hbm.at[pl.ds(0, CROWS)], fbufs[b], isems[b]).wait()

            @pl.when(k > 0)
            def _():
                pltpu.make_async_copy(
                    obufs[b], out_hbm.at[pl.ds(0, CROWS)], osems[b]).wait()
            pack_chunk(b)
            pltpu.async_copy(
                obufs[b], out_hbm.at[pl.ds(rbase + kk * CROWS, CROWS)],
                osems[b])

            @pl.when(kk + 2 < NCGRP)
            def _():
                pltpu.async_copy(
                    emb_hbm.at[pl.ds(rbase + (kk + 2) * CROWS, CROWS)],
                    fbufs[b], isems[b])
        return carry

    lax.fori_loop(0, NCGRP // 2, grp_body, 0)
    pltpu.make_async_copy(
        emb_hbm.at[pl.ds(0, CROWS)], fbufs[0], isems[0]).wait()
    pltpu.make_async_copy(
        obufs[0], out_hbm.at[pl.ds(0, CROWS)], osems[0]).wait()
    pack_chunk(0)
    pltpu.async_copy(
        obufs[0], out_hbm.at[pl.ds(rbase + (NCGRP - 1) * CROWS, CROWS)],
        osems[0])
    pltpu.make_async_copy(
        obufs[0], out_hbm.at[pl.ds(0, CROWS)], osems[0]).wait()
    pltpu.make_async_copy(
        obufs[1], out_hbm.at[pl.ds(0, CROWS)], osems[1]).wait()


@functools.lru_cache(maxsize=1)
def _make_conv():
    return pl.kernel(
        _conv_body,
        out_type=jax.ShapeDtypeStruct((VOCAB, EMB_DIM), jnp.bfloat16),
        scratch_types=(
            [pltpu.VMEM((CROWS, EMB_DIM), jnp.float32) for _ in range(2)]
            + [pltpu.VMEM((CROWS, EMB_DIM), jnp.bfloat16) for _ in range(2)]
            + [pltpu.SemaphoreType.DMA for _ in range(4)]
        ),
        **_SC_PARAMS,
    )


def _pool_body(idxa_hbm, idxb_hbm, emb_hbm, out_hbm, idx_va, idx_vb,
               b0, b1, b2, b3, acc_v, s0, s1, s2, s3):
    bufs = (b0, b1, b2, b3)
    sems = (s0, s1, s2, s3)
    idx_vs = (idx_va, idx_vb)
    wid = lax.axis_index("s") * NC + lax.axis_index("c")
    rbase = wid * ROWS_PER_W
    pltpu.sync_copy(idxa_hbm.at[pl.ds(rbase, ROWS_PER_W)], idx_va)
    pltpu.sync_copy(idxb_hbm.at[pl.ds(rbase, ROWS_PER_W)], idx_vb)

    for b in range(NBUF):
        pltpu.async_copy(
            emb_hbm.at[idx_vs[b % 2].at[b // 2]], bufs[b], sems[b])

    def acc_chunk(buf, chains):
        for t in range(CHUNK):
            for h in range(NH):
                x = buf[t, pl.ds(32 * h, 32)]
                a, c = plsc.unpack(x, format=plsc.PackFormat.INTERLEAVED)
                hv = 0 if t < HALF else 1
                for ab, v in ((0, a), (1, c)):
                    cur = chains[h][ab][hv]
                    chains[h][ab][hv] = v if cur is None else cur + v
        return chains

    def grp_body(g, carry):
        for p in range(NBUF // 2):
            chains = [[[None, None], [None, None]] for _ in range(NH)]
            for q in range(2):
                b = 2 * p + q
                pltpu.make_async_copy(
                    emb_hbm.at[idx_va.at[0]], bufs[b], sems[b]).wait()
                chains = acc_chunk(bufs[b], chains)

                @pl.when(g < NGRP - 1)
                def _():
                    pltpu.async_copy(
                        emb_hbm.at[idx_vs[q].at[2 * g + p + 2]],
                        bufs[b], sems[b])
            r = (NBUF // 2) * g + p
            for h in range(NH):
                for ab in range(2):
                    tot = chains[h][ab][0] + chains[h][ab][1]
                    acc_v[r, pl.ds(32 * h + 16 * ab, 16)] = tot
        return carry

    lax.fori_loop(0, NGRP, grp_body, 0)
    pltpu.sync_copy(acc_v, out_hbm.at[pl.ds(wid * ROWS_PER_W, ROWS_PER_W)])


@functools.lru_cache(maxsize=1)
def _make_pool():
    return pl.kernel(
        _pool_body,
        out_type=jax.ShapeDtypeStruct((B, EMB_DIM), jnp.float32),
        scratch_types=(
            [pltpu.VMEM((ROWS_PER_W, CHUNK), jnp.int32) for _ in range(2)]
            + [pltpu.VMEM((CHUNK, EMB_DIM), jnp.bfloat16)
               for _ in range(NBUF)]
            + [pltpu.VMEM((ROWS_PER_W, EMB_DIM), jnp.float32)]
            + [pltpu.SemaphoreType.DMA for _ in range(NBUF)]
        ),
        **_SC_PARAMS,
    )


def _mlp_body(x_ref, w1t_ref, b1_ref, w2t_ref, b2_ref, o_ref):
    x = x_ref[:] * (1.0 / L)
    h = jnp.dot(x, w1t_ref[:], preferred_element_type=jnp.float32) + b1_ref[:]
    h = jnp.maximum(h, 0.0)
    o = jnp.dot(h, w2t_ref[:], preferred_element_type=jnp.float32) + b2_ref[:]
    m = jnp.max(o, axis=1, keepdims=True)
    lse = jnp.log(jnp.sum(jnp.exp(o - m), axis=1, keepdims=True)) + m
    o_ref[:] = o - lse


def _mlp(sums, w1t, b1_2d, w2tp, b2p):
    blk = B // 4
    return pl.pallas_call(
        _mlp_body,
        grid=(4,),
        in_specs=[
            pl.BlockSpec((blk, EMB_DIM), lambda i: (i, 0)),
            pl.BlockSpec((EMB_DIM, HIDDEN), lambda i: (0, 0)),
            pl.BlockSpec((1, HIDDEN), lambda i: (0, 0)),
            pl.BlockSpec((HIDDEN, OUT_PAD), lambda i: (0, 0)),
            pl.BlockSpec((1, OUT_PAD), lambda i: (0, 0)),
        ],
        out_specs=pl.BlockSpec((blk, OUT_PAD), lambda i: (i, 0)),
        out_shape=jax.ShapeDtypeStruct((B, OUT_PAD), jnp.float32),
    )(sums, w1t, b1_2d, w2tp, b2p)


def kernel(subword_indices, emb, W1, b1, W2, b2):
    idx = subword_indices.astype(jnp.int32)
    emb_packed = _make_conv()(emb)
    sums = _make_pool()(idx[:, :CHUNK], idx[:, CHUNK:], emb_packed)
    w1t = W1.T
    b1_2d = b1.reshape(1, HIDDEN)
    w2tp = jnp.zeros((HIDDEN, OUT_PAD), jnp.float32).at[:, :OUT].set(W2.T)
    b2p = jnp.full((1, OUT_PAD), -1e30, jnp.float32).at[0, :OUT].set(b2)
    out = _mlp(sums, w1t, b1_2d, w2tp, b2p)
    return out[:, :OUT]
